# baseline (speedup 1.0000x reference)
import functools

import jax
import jax.numpy as jnp
from jax import lax
from jax.experimental import pallas as pl
from jax.experimental.pallas import tpu as pltpu
from jax.experimental.pallas import tpu_sc as plsc

NP = 50000
NA = 20000
NP_PAD = 51200
NA_PAD = 20480
PAD_IDX = NP_PAD - 1
NT = 16
LANE = 128
BLK = 1024
E_PAD = 20480
CHUNKS = E_PAD // LANE
KP = 5
GP = CHUNKS // (2 * KP)
SL = NP_PAD // NT

_sc_mesh = plsc.VectorSubcoreMesh(core_axis_name="c", subcore_axis_name="s")


@functools.partial(
    pl.kernel,
    out_type=jax.ShapeDtypeStruct((2, NP_PAD), jnp.float32),
    mesh=_sc_mesh,
    scratch_types=[
        pltpu.VMEM((CHUNKS, LANE), jnp.int32),
        pltpu.VMEM((LANE,), jnp.float32),
        pltpu.SemaphoreType.DMA,
        pltpu.VMEM_SHARED((NP_PAD,), jnp.float32),
    ],
)
def _sc_count(dsts_hbm, zeros_hbm, ones_hbm, cnt_out, idx_v, ones_v, sem, acc):
    c = lax.axis_index("c")
    s = lax.axis_index("s")
    w = c * NT + s
    pltpu.sync_copy(zeros_hbm.at[pl.ds(s * SL, SL)], acc.at[pl.ds(s * SL, SL)])
    pltpu.sync_copy(ones_hbm, ones_v)
    pltpu.sync_copy(dsts_hbm.at[w], idx_v)
    plsc.subcore_barrier()

    def body(gi, carry):
        base = gi * 2 * KP
        for k in range(2 * KP):
            pltpu.async_copy(ones_v, acc.at[idx_v.at[base + k]], sem, add=True)
        for k in range(2 * KP):
            pltpu.make_async_copy(ones_v, acc.at[idx_v.at[base + k]], sem).wait()
        return carry

    lax.fori_loop(0, GP, body, 0)
    plsc.subcore_barrier()
    pltpu.sync_copy(acc.at[pl.ds(s * SL, SL)], cnt_out.at[c].at[pl.ds(s * SL, SL)])


@functools.partial(
    pl.kernel,
    out_type=jax.ShapeDtypeStruct((2, NP_PAD), jnp.float32),
    mesh=_sc_mesh,
    scratch_types=[
        pltpu.VMEM((CHUNKS, LANE), jnp.int32),
        pltpu.VMEM((CHUNKS, LANE), jnp.int32),
        pltpu.VMEM((2 * KP, LANE), jnp.float32),
        pltpu.SemaphoreType.DMA,
        pltpu.SemaphoreType.DMA,
        pltpu.SemaphoreType.DMA,
        pltpu.VMEM_SHARED((NP_PAD,), jnp.float32),
    ],
)
def _sc_scatter(srcs_hbm, dsts_hbm, table_hbm, zeros_hbm, acc_out,
                src_v, dst_v, vals_v, gsem0, gsem1, ssem, acc):
    c = lax.axis_index("c")
    s = lax.axis_index("s")
    w = c * NT + s
    pltpu.sync_copy(zeros_hbm.at[pl.ds(s * SL, SL)], acc.at[pl.ds(s * SL, SL)])
    pltpu.sync_copy(srcs_hbm.at[w], src_v)
    pltpu.sync_copy(dsts_hbm.at[w], dst_v)
    plsc.subcore_barrier()

    def gather(j, buf, sem):
        return pltpu.async_copy(table_hbm.at[src_v.at[j]], vals_v.at[buf], sem)

    def gather_wait(j, buf, sem):
        pltpu.make_async_copy(table_hbm.at[src_v.at[j]], vals_v.at[buf],
                              sem).wait()

    def body(p, carry):
        b0 = 2 * p * KP
        b1 = b0 + KP
        for k in range(KP):
            gather_wait(b0 + k, k, gsem0)
        for k in range(KP):
            gather(b1 + k, KP + k, gsem1)
        for k in range(KP):
            pltpu.async_copy(vals_v.at[k], acc.at[dst_v.at[b0 + k]], ssem,
                             add=True)
        for k in range(KP):
            pltpu.make_async_copy(vals_v.at[k], acc.at[dst_v.at[b0 + k]],
                                  ssem).wait()

        @pl.when(p + 1 < GP)
        def _():
            for k in range(KP):
                gather(b1 + KP + k, k, gsem0)

        for k in range(KP):
            gather_wait(b1 + k, KP + k, gsem1)
        for k in range(KP):
            pltpu.async_copy(vals_v.at[KP + k], acc.at[dst_v.at[b1 + k]], ssem,
                             add=True)
        for k in range(KP):
            pltpu.make_async_copy(vals_v.at[KP + k], acc.at[dst_v.at[b1 + k]],
                                  ssem).wait()
        return carry

    for k in range(KP):
        gather(k, k, gsem0)
    lax.fori_loop(0, GP, body, 0)
    plsc.subcore_barrier()
    pltpu.sync_copy(acc.at[pl.ds(s * SL, SL)], acc_out.at[c].at[pl.ds(s * SL, SL)])


TR = BLK // LANE


def _tc_pre_body(xp_ref, xa_ref, deg_ref, cnt_ref, wg_ref, wl_ref, wr_ref,
                 wo_ref, bg_ref, bs_ref, bo_ref,
                 h_ref, dis_ref, invc_ref, s_ref, a_ref):
    i = pl.program_id(0)
    wo = wo_ref[...]
    x = xp_ref[...]
    g = jnp.dot(x, wg_ref[...] @ wo,
                preferred_element_type=jnp.float32).reshape(TR, LANE)
    r = jnp.dot(x, wr_ref[...] @ wo,
                preferred_element_type=jnp.float32).reshape(TR, LANE)
    deg = deg_ref[...] + 1.0
    dis = lax.rsqrt(deg)
    cval = (bg_ref[...] + bs_ref[...]) @ wo + bo_ref[...]
    h_ref[...] = g * dis
    dis_ref[...] = dis
    invc_ref[...] = 1.0 / jnp.maximum(cnt_ref[...], 1.0)
    s_ref[...] = g / deg + r + cval

    @pl.when(i < NA_PAD // BLK)
    def _():
        a_ref[...] = jnp.dot(xa_ref[...], wl_ref[...] @ wo,
                             preferred_element_type=jnp.float32
                             ).reshape(TR, LANE)


def _tc_pre(xp, xa, deg, cnt, wg, wl, wr, wo, bg, bs, bo):
    n_a = NA_PAD // BLK
    n_p = NP // BLK
    grid = (NP_PAD // BLK,)
    tile = lambda i: (i, 0)
    pcol = lambda i: (jnp.minimum(i, n_p), 0)
    acol = lambda i: (jnp.minimum(i, n_a - 1), 0)
    fixed = lambda i: (0, 0)
    tspec = pl.BlockSpec((TR, LANE), tile)
    return pl.pallas_call(
        _tc_pre_body,
        grid=grid,
        in_specs=[
            pl.BlockSpec((BLK, 128), pcol),
            pl.BlockSpec((BLK, 128), acol),
            tspec,
            tspec,
            pl.BlockSpec((128, 32), fixed),
            pl.BlockSpec((128, 32), fixed),
            pl.BlockSpec((128, 32), fixed),
            pl.BlockSpec((32, 1), fixed),
            pl.BlockSpec((1, 32), fixed),
            pl.BlockSpec((1, 32), fixed),
            pl.BlockSpec((1, 1), fixed),
        ],
        out_specs=[
            tspec,
            tspec,
            tspec,
            tspec,
            pl.BlockSpec((TR, LANE), acol),
        ],
        out_shape=[
            jax.ShapeDtypeStruct((NP_PAD // LANE, LANE), jnp.float32),
            jax.ShapeDtypeStruct((NP_PAD // LANE, LANE), jnp.float32),
            jax.ShapeDtypeStruct((NP_PAD // LANE, LANE), jnp.float32),
            jax.ShapeDtypeStruct((NP_PAD // LANE, LANE), jnp.float32),
            jax.ShapeDtypeStruct((NA_PAD // LANE, LANE), jnp.float32),
        ],
    )(xp, xa, deg, cnt, wg, wl, wr, wo, bg, bs, bo)


def _tc_combine_body(dis_ref, invc_ref, s_ref, app_ref, aap_ref, o_ref):
    o_ref[...] = (dis_ref[...] * app_ref[...]
                  + invc_ref[...] * aap_ref[...] + s_ref[...])


def _tc_combine(dis, invc, s, app, aap):
    spec = pl.BlockSpec((TR, LANE), lambda i: (i, 0))
    return pl.pallas_call(
        _tc_combine_body,
        grid=(NP_PAD // BLK,),
        in_specs=[spec] * 5,
        out_specs=spec,
        out_shape=jax.ShapeDtypeStruct((NP_PAD // LANE, LANE), jnp.float32),
    )(dis, invc, s, app, aap)


def _prep_edges(ei, src_off):
    e = ei.shape[1]
    pad = NT * E_PAD - e
    src = jnp.concatenate(
        [ei[0] + src_off, jnp.zeros((pad,), jnp.int32) + src_off])
    dst = jnp.concatenate([ei[1], jnp.full((pad,), PAD_IDX, jnp.int32)])
    return src.reshape(NT, CHUNKS, LANE), dst.reshape(NT, CHUNKS, LANE)


def kernel(x_paper, x_author, edge_index_paper, edge_index_author,
           W_gcn, b_gcn, W_sage_l, W_sage_r, b_sage, W_out, b_out):
    src_pp, dst_pp = _prep_edges(edge_index_paper, 0)
    src_ap, dst_ap = _prep_edges(edge_index_author, NP_PAD)
    srcs = jnp.concatenate([src_pp, src_ap]).reshape(2 * NT, CHUNKS, LANE)
    dsts = jnp.concatenate([dst_pp, dst_ap]).reshape(2 * NT, CHUNKS, LANE)
    zeros = jnp.zeros((NP_PAD,), jnp.float32)
    ones = jnp.ones((LANE,), jnp.float32)

    counts = _sc_count(dsts, zeros, ones)
    deg_m = counts[0].reshape(NP_PAD // LANE, LANE)
    cnt = counts[1].reshape(NP_PAD // LANE, LANE)

    h, dis, invc, s, a = _tc_pre(
        x_paper, x_author, deg_m, cnt, W_gcn, W_sage_l, W_sage_r, W_out,
        b_gcn.reshape(1, 32), b_sage.reshape(1, 32), b_out.reshape(1, 1))

    table = jnp.concatenate([h.reshape(NP_PAD), a.reshape(NA_PAD)])
    accs = _sc_scatter(srcs, dsts, table, zeros)
    app = accs[0].reshape(NP_PAD // LANE, LANE)
    aap = accs[1].reshape(NP_PAD // LANE, LANE)

    out = _tc_combine(dis, invc, s, app, aap)
    return out.reshape(NP_PAD, 1)[:NP]


# speedup vs baseline: 1.3827x; 1.3827x over previous
import functools

import jax
import jax.numpy as jnp
from jax import lax
from jax.experimental import pallas as pl
from jax.experimental.pallas import tpu as pltpu
from jax.experimental.pallas import tpu_sc as plsc

NP = 50000
NA = 20000
NP_PAD = 51200
NA_PAD = 20480
PAD_IDX = NP_PAD - 1
NT = 16
LANE = 128
BLK = 1024
E_PAD = 19456
CHUNKS = E_PAD // LANE
KP = 4
GP = CHUNKS // (2 * KP)
SL = NP_PAD // NT

_sc_mesh = plsc.VectorSubcoreMesh(core_axis_name="c", subcore_axis_name="s")


@functools.partial(
    pl.kernel,
    out_type=jax.ShapeDtypeStruct((2, NP_PAD), jnp.float32),
    mesh=_sc_mesh,
    scratch_types=[
        pltpu.VMEM((CHUNKS, LANE), jnp.int32),
        pltpu.VMEM((LANE,), jnp.float32),
        pltpu.SemaphoreType.DMA,
        pltpu.VMEM_SHARED((NP_PAD,), jnp.float32),
    ],
)
def _sc_count(dsts_hbm, zeros_hbm, ones_hbm, cnt_out, idx_v, ones_v, sem, acc):
    c = lax.axis_index("c")
    s = lax.axis_index("s")
    w = c * NT + s
    pltpu.sync_copy(zeros_hbm.at[pl.ds(s * SL, SL)], acc.at[pl.ds(s * SL, SL)])
    pltpu.sync_copy(ones_hbm, ones_v)
    pltpu.sync_copy(dsts_hbm.at[w], idx_v)
    plsc.subcore_barrier()

    def body(gi, carry):
        base = gi * 2 * KP
        for k in range(2 * KP):
            pltpu.async_copy(ones_v, acc.at[idx_v.at[base + k]], sem, add=True)
        for k in range(2 * KP):
            pltpu.make_async_copy(ones_v, acc.at[idx_v.at[base + k]], sem).wait()
        return carry

    lax.fori_loop(0, GP, body, 0)
    plsc.subcore_barrier()
    pltpu.sync_copy(acc.at[pl.ds(s * SL, SL)], cnt_out.at[c].at[pl.ds(s * SL, SL)])


@functools.partial(
    pl.kernel,
    out_type=jax.ShapeDtypeStruct((2, NP_PAD), jnp.float32),
    mesh=_sc_mesh,
    scratch_types=[
        pltpu.VMEM((CHUNKS, LANE), jnp.int32),
        pltpu.VMEM((CHUNKS, LANE), jnp.int32),
        pltpu.VMEM((2 * KP, LANE), jnp.float32),
        pltpu.SemaphoreType.DMA,
        pltpu.SemaphoreType.DMA,
        pltpu.SemaphoreType.DMA,
        pltpu.VMEM_SHARED((NP_PAD,), jnp.float32),
    ],
)
def _sc_scatter(srcs_hbm, dsts_hbm, table_hbm, zeros_hbm, acc_out,
                src_v, dst_v, vals_v, gsem0, gsem1, ssem, acc):
    c = lax.axis_index("c")
    s = lax.axis_index("s")
    w = c * NT + s
    pltpu.sync_copy(zeros_hbm.at[pl.ds(s * SL, SL)], acc.at[pl.ds(s * SL, SL)])
    pltpu.sync_copy(srcs_hbm.at[w], src_v)
    pltpu.sync_copy(dsts_hbm.at[w], dst_v)
    plsc.subcore_barrier()

    def gather(j, buf, sem):
        return pltpu.async_copy(table_hbm.at[src_v.at[j]], vals_v.at[buf], sem)

    def gather_wait(j, buf, sem):
        pltpu.make_async_copy(table_hbm.at[src_v.at[j]], vals_v.at[buf],
                              sem).wait()

    def body(p, carry):
        b0 = 2 * p * KP
        b1 = b0 + KP
        for k in range(KP):
            gather_wait(b0 + k, k, gsem0)
        for k in range(KP):
            gather(b1 + k, KP + k, gsem1)
        for k in range(KP):
            pltpu.async_copy(vals_v.at[k], acc.at[dst_v.at[b0 + k]], ssem,
                             add=True)
        for k in range(KP):
            pltpu.make_async_copy(vals_v.at[k], acc.at[dst_v.at[b0 + k]],
                                  ssem).wait()

        @pl.when(p + 1 < GP)
        def _():
            for k in range(KP):
                gather(b1 + KP + k, k, gsem0)

        for k in range(KP):
            gather_wait(b1 + k, KP + k, gsem1)
        for k in range(KP):
            pltpu.async_copy(vals_v.at[KP + k], acc.at[dst_v.at[b1 + k]], ssem,
                             add=True)
        for k in range(KP):
            pltpu.make_async_copy(vals_v.at[KP + k], acc.at[dst_v.at[b1 + k]],
                                  ssem).wait()
        return carry

    for k in range(KP):
        gather(k, k, gsem0)
    lax.fori_loop(0, GP, body, 0)
    plsc.subcore_barrier()
    pltpu.sync_copy(acc.at[pl.ds(s * SL, SL)], acc_out.at[c].at[pl.ds(s * SL, SL)])


TR = BLK // LANE


def _tc_pre_body(xp_ref, xa_ref, deg_ref, cnt_ref, wg_ref, wl_ref, wr_ref,
                 wo_ref, bg_ref, bs_ref, bo_ref,
                 h_ref, dis_ref, invc_ref, s_ref, a_ref):
    i = pl.program_id(0)
    wo = wo_ref[...]
    x = xp_ref[...]
    g = jnp.dot(x, wg_ref[...] @ wo,
                preferred_element_type=jnp.float32).reshape(TR, LANE)
    r = jnp.dot(x, wr_ref[...] @ wo,
                preferred_element_type=jnp.float32).reshape(TR, LANE)
    deg = deg_ref[...] + 1.0
    dis = lax.rsqrt(deg)
    cval = (bg_ref[...] + bs_ref[...]) @ wo + bo_ref[...]
    h_ref[...] = g * dis
    dis_ref[...] = dis
    invc_ref[...] = 1.0 / jnp.maximum(cnt_ref[...], 1.0)
    s_ref[...] = g / deg + r + cval

    @pl.when(i < NA_PAD // BLK)
    def _():
        a_ref[...] = jnp.dot(xa_ref[...], wl_ref[...] @ wo,
                             preferred_element_type=jnp.float32
                             ).reshape(TR, LANE)


def _tc_pre(xp, xa, deg, cnt, wg, wl, wr, wo, bg, bs, bo):
    n_a = NA_PAD // BLK
    n_p = NP // BLK
    grid = (NP_PAD // BLK,)
    tile = lambda i: (i, 0)
    pcol = lambda i: (jnp.minimum(i, n_p), 0)
    acol = lambda i: (jnp.minimum(i, n_a - 1), 0)
    fixed = lambda i: (0, 0)
    tspec = pl.BlockSpec((TR, LANE), tile)
    return pl.pallas_call(
        _tc_pre_body,
        grid=grid,
        in_specs=[
            pl.BlockSpec((BLK, 128), pcol),
            pl.BlockSpec((BLK, 128), acol),
            tspec,
            tspec,
            pl.BlockSpec((128, 32), fixed),
            pl.BlockSpec((128, 32), fixed),
            pl.BlockSpec((128, 32), fixed),
            pl.BlockSpec((32, 1), fixed),
            pl.BlockSpec((1, 32), fixed),
            pl.BlockSpec((1, 32), fixed),
            pl.BlockSpec((1, 1), fixed),
        ],
        out_specs=[
            tspec,
            tspec,
            tspec,
            tspec,
            pl.BlockSpec((TR, LANE), acol),
        ],
        out_shape=[
            jax.ShapeDtypeStruct((NP_PAD // LANE, LANE), jnp.float32),
            jax.ShapeDtypeStruct((NP_PAD // LANE, LANE), jnp.float32),
            jax.ShapeDtypeStruct((NP_PAD // LANE, LANE), jnp.float32),
            jax.ShapeDtypeStruct((NP_PAD // LANE, LANE), jnp.float32),
            jax.ShapeDtypeStruct((NA_PAD // LANE, LANE), jnp.float32),
        ],
    )(xp, xa, deg, cnt, wg, wl, wr, wo, bg, bs, bo)


def _tc_combine_body(dis_ref, invc_ref, s_ref, app_ref, aap_ref, o_ref):
    o_ref[...] = (dis_ref[...] * app_ref[...]
                  + invc_ref[...] * aap_ref[...] + s_ref[...])


def _tc_combine(dis, invc, s, app, aap):
    spec = pl.BlockSpec((TR, LANE), lambda i: (i, 0))
    return pl.pallas_call(
        _tc_combine_body,
        grid=(NP_PAD // BLK,),
        in_specs=[spec] * 5,
        out_specs=spec,
        out_shape=jax.ShapeDtypeStruct((NP_PAD // LANE, LANE), jnp.float32),
    )(dis, invc, s, app, aap)


def _prep_edges(ei, src_off):
    e = ei.shape[1]
    pad = NT * E_PAD - e
    src = jnp.concatenate(
        [ei[0] + src_off, jnp.zeros((pad,), jnp.int32) + src_off])
    dst = jnp.concatenate([ei[1], jnp.full((pad,), PAD_IDX, jnp.int32)])
    return src.reshape(NT, CHUNKS, LANE), dst.reshape(NT, CHUNKS, LANE)


def kernel(x_paper, x_author, edge_index_paper, edge_index_author,
           W_gcn, b_gcn, W_sage_l, W_sage_r, b_sage, W_out, b_out):
    src_pp, dst_pp = _prep_edges(edge_index_paper, 0)
    src_ap, dst_ap = _prep_edges(edge_index_author, NP_PAD)
    srcs = jnp.concatenate([src_pp, src_ap]).reshape(2 * NT, CHUNKS, LANE)
    dsts = jnp.concatenate([dst_pp, dst_ap]).reshape(2 * NT, CHUNKS, LANE)
    zeros = jnp.zeros((NP_PAD,), jnp.float32)
    ones = jnp.ones((LANE,), jnp.float32)

    counts = _sc_count(dsts, zeros, ones)
    deg_m = counts[0].reshape(NP_PAD // LANE, LANE)
    cnt = counts[1].reshape(NP_PAD // LANE, LANE)

    h, dis, invc, s, a = _tc_pre(
        x_paper, x_author, deg_m, cnt, W_gcn, W_sage_l, W_sage_r, W_out,
        b_gcn.reshape(1, 32), b_sage.reshape(1, 32), b_out.reshape(1, 1))

    table = jnp.concatenate([h.reshape(NP_PAD), a.reshape(NA_PAD)])
    accs = _sc_scatter(srcs, dsts, table, zeros)
    app = accs[0].reshape(NP_PAD // LANE, LANE)
    aap = accs[1].reshape(NP_PAD // LANE, LANE)

    out = _tc_combine(dis, invc, s, app, aap)
    return out.reshape(NP_PAD, 1)[:NP]
